# baseline (speedup 1.0000x reference)
import jax
import jax.numpy as jnp
from jax.experimental import pallas as pl


def _body(x_ref, w1_ref, b1_ref, w2_ref, b2_ref, u_ref, out_ref):
    h = jnp.tanh(
        jnp.dot(x_ref[...], w1_ref[...], preferred_element_type=jnp.float32)
        + b1_ref[...]
    )
    p4 = jnp.dot(h, w2_ref[...], preferred_element_type=jnp.float32) + b2_ref[...]
    l0 = p4[:, 0]
    l1 = p4[:, 1]
    mu = p4[:, 2]
    s_raw = p4[:, 3]
    m = jnp.maximum(l0, l1)
    e0 = jnp.exp(l0 - m)
    e1 = jnp.exp(l1 - m)
    p_d = e0 / (e0 + e1)
    s = jax.nn.softplus(s_raw)
    p_rain = u_ref[0, :]
    p_dist = u_ref[1, :]
    ppf = jnp.exp(mu + s * (jnp.log(p_dist) - jnp.log1p(-p_dist)))
    out_ref[...] = jnp.where(p_rain <= p_d, jnp.float32(0.0), ppf)


def kernel(x, W1, b1, W2, b2, u):
    n, d_in = x.shape
    d_h = W1.shape[1]
    B = 2048
    grid = (n // B,)
    return pl.pallas_call(
        _body,
        grid=grid,
        in_specs=[
            pl.BlockSpec((B, d_in), lambda i: (i, 0)),
            pl.BlockSpec((d_in, d_h), lambda i: (0, 0)),
            pl.BlockSpec((d_h,), lambda i: (0,)),
            pl.BlockSpec((d_h, 4), lambda i: (0, 0)),
            pl.BlockSpec((4,), lambda i: (0,)),
            pl.BlockSpec((2, B), lambda i: (0, i)),
        ],
        out_specs=pl.BlockSpec((B,), lambda i: (i,)),
        out_shape=jax.ShapeDtypeStruct((n,), jnp.float32),
    )(x, W1, b1, W2, b2, u)


# speedup vs baseline: 2.6820x; 2.6820x over previous
import jax
import jax.numpy as jnp
from jax import lax
from jax.experimental import pallas as pl


def _body(x_ref, w1_ref, b1_ref, w2_ref, b2_ref, u_ref, out_ref):
    h = jnp.tanh(
        jnp.dot(x_ref[...], w1_ref[...], preferred_element_type=jnp.float32)
        + b1_ref[...]
    )
    p4t = lax.dot_general(
        w2_ref[...], h, (((0,), (1,)), ((), ())),
        preferred_element_type=jnp.float32,
    ) + b2_ref[...].reshape(4, 1)
    l0 = p4t[0:1, :]
    l1 = p4t[1:2, :]
    mu = p4t[2:3, :]
    s_raw = p4t[3:4, :]
    m = jnp.maximum(l0, l1)
    e0 = jnp.exp(l0 - m)
    e1 = jnp.exp(l1 - m)
    p_d = e0 / (e0 + e1)
    s = jax.nn.softplus(s_raw)
    p_rain = u_ref[0:1, :]
    p_dist = u_ref[1:2, :]
    ppf = jnp.exp(mu + s * (jnp.log(p_dist) - jnp.log1p(-p_dist)))
    out_ref[...] = jnp.where(p_rain <= p_d, jnp.float32(0.0), ppf)


def kernel(x, W1, b1, W2, b2, u):
    n, d_in = x.shape
    d_h = W1.shape[1]
    B = 2048
    grid = (n // B,)
    out = pl.pallas_call(
        _body,
        grid=grid,
        in_specs=[
            pl.BlockSpec((B, d_in), lambda i: (i, 0)),
            pl.BlockSpec((d_in, d_h), lambda i: (0, 0)),
            pl.BlockSpec((d_h,), lambda i: (0,)),
            pl.BlockSpec((d_h, 4), lambda i: (0, 0)),
            pl.BlockSpec((4,), lambda i: (0,)),
            pl.BlockSpec((2, B), lambda i: (0, i)),
        ],
        out_specs=pl.BlockSpec((1, B), lambda i: (0, i)),
        out_shape=jax.ShapeDtypeStruct((1, n), jnp.float32),
    )(x, W1, b1, W2, b2, u)
    return out.reshape(n)


# speedup vs baseline: 3.1348x; 1.1688x over previous
import jax
import jax.numpy as jnp
from jax import lax
from jax.experimental import pallas as pl

_P = 4
_B = 1024


def _tail(p4t, u):
    l0 = p4t[0:1, :]
    l1 = p4t[1:2, :]
    mu = p4t[2:3, :]
    s_raw = p4t[3:4, :]
    m = jnp.maximum(l0, l1)
    e0 = jnp.exp(l0 - m)
    e1 = jnp.exp(l1 - m)
    p_d = e0 / (e0 + e1)
    s = jax.nn.softplus(s_raw)
    p_rain = u[0:1, :]
    p_dist = u[1:2, :]
    ppf = jnp.exp(mu + s * (jnp.log(p_dist) - jnp.log1p(-p_dist)))
    return jnp.where(p_rain <= p_d, jnp.float32(0.0), ppf)


def _body(*refs):
    x_refs = refs[:_P]
    w1_ref, b1_ref, w2_ref, b2_ref, u_ref, out_ref = refs[_P:]
    w1 = w1_ref[...]
    w2 = w2_ref[...]
    b1 = b1_ref[...]
    b2c = b2_ref[...].reshape(4, 1)
    for p in range(_P):
        h = jnp.tanh(
            jnp.dot(x_refs[p][...], w1, preferred_element_type=jnp.float32) + b1
        )
        p4t = lax.dot_general(
            w2, h, (((0,), (1,)), ((), ())),
            preferred_element_type=jnp.float32,
        ) + b2c
        u_p = u_ref[:, p * _B:(p + 1) * _B]
        out_ref[:, p * _B:(p + 1) * _B] = _tail(p4t, u_p)


def kernel(x, W1, b1, W2, b2, u):
    n, d_in = x.shape
    d_h = W1.shape[1]
    rows_per_step = _P * _B
    grid = (n // rows_per_step,)
    x_specs = [
        pl.BlockSpec((_B, d_in), lambda i, p=p: (i * _P + p, 0)) for p in range(_P)
    ]
    out = pl.pallas_call(
        _body,
        grid=grid,
        in_specs=x_specs + [
            pl.BlockSpec((d_in, d_h), lambda i: (0, 0)),
            pl.BlockSpec((d_h,), lambda i: (0,)),
            pl.BlockSpec((d_h, 4), lambda i: (0, 0)),
            pl.BlockSpec((4,), lambda i: (0,)),
            pl.BlockSpec((2, rows_per_step), lambda i: (0, i)),
        ],
        out_specs=pl.BlockSpec((1, rows_per_step), lambda i: (0, i)),
        out_shape=jax.ShapeDtypeStruct((1, n), jnp.float32),
    )(*([x] * _P), W1, b1, W2, b2, u)
    return out.reshape(n)
